# baseline (speedup 1.0000x reference)
import jax
import jax.numpy as jnp
from jax import lax
from jax.experimental import pallas as pl
from jax.experimental.pallas import tpu as pltpu
from jax.experimental.pallas import tpu_sc as plsc

N_NODES = 10000
N_HEDGES = 20000
NNZ = 320000
D = 128
Q = 32
NC = 2
NS = 16
CH = 128
NNZP = 327680
RPT = NNZP // NS // CH
VG = N_NODES
EG = N_HEDGES
XN = 10016
EN = 20480
VN = 10016
ERT = EN // NS
VRT = VN // NS

_MESH = dict(core_axis_name="c", subcore_axis_name="s", num_cores=NC,
             num_subcores=NS)
_PARAMS = pltpu.CompilerParams(use_tc_tiling_on_sc=False)


def _pipelined_pass(table, idx_g, idx_s, rows, acc, semg, sems, hook=None):
    pltpu.async_copy(table.at[idx_g.at[0]], rows.at[0], semg.at[0])

    def body(j, carry):
        nxt = j + 1

        @pl.when(nxt < RPT)
        def _():
            @pl.when(j >= 1)
            def _():
                pltpu.make_async_copy(
                    rows.at[nxt % 2], acc.at[idx_s.at[j - 1]],
                    sems.at[nxt % 2]).wait()

            pltpu.async_copy(table.at[idx_g.at[nxt]], rows.at[nxt % 2],
                             semg.at[nxt % 2])

        pltpu.make_async_copy(table.at[idx_g.at[j]], rows.at[j % 2],
                              semg.at[j % 2]).wait()
        pltpu.async_copy(rows.at[j % 2], acc.at[idx_s.at[j]],
                         sems.at[j % 2], add=True)
        if hook is not None:
            hook(j)
        return carry

    lax.fori_loop(0, RPT, body, 0)
    pltpu.make_async_copy(rows.at[0], acc.at[idx_s.at[RPT - 2]],
                          sems.at[0]).wait()
    pltpu.make_async_copy(rows.at[1], acc.at[idx_s.at[RPT - 1]],
                          sems.at[1]).wait()


def _phase1_body(xs, vv, ee, zq, z16, ones_h, xe_out, deg_out,
                 vidx, eidx, rows, ones_v, xe_sh, deg_sh, semg, sems):
    c = lax.axis_index("c")
    s = lax.axis_index("s")
    pltpu.sync_copy(ones_h, ones_v)
    pltpu.sync_copy(vv.at[s], vidx)
    pltpu.sync_copy(ee.at[s], eidx)
    pltpu.sync_copy(z16, deg_sh.at[pl.ds(s * VRT, VRT)])
    r0 = s * ERT
    half = RPT // 2
    for k in range(2):
        g = 2 * c + k
        pltpu.sync_copy(zq, xe_sh.at[pl.ds(r0, ERT // 2)])
        pltpu.sync_copy(zq, xe_sh.at[pl.ds(r0 + ERT // 2, ERT // 2)])
        plsc.subcore_barrier()

        def deg_hook(j):
            if k == 0:
                @pl.when(jnp.logical_and(j >= c * half, j < (c + 1) * half))
                def _():
                    pltpu.sync_copy(ones_v, deg_sh.at[vidx.at[j]], add=True)

        _pipelined_pass(xs.at[g], vidx, eidx, rows, xe_sh, semg, sems,
                        hook=deg_hook)
        plsc.subcore_barrier()
        pltpu.sync_copy(xe_sh.at[pl.ds(r0, ERT)],
                        xe_out.at[c, s, :, pl.ds(k * Q, Q)])
    pltpu.sync_copy(deg_sh.at[pl.ds(s * VRT, VRT)], deg_out.at[c].at[s])


def _phase2_body(xe2, vv, ee, zh, xv_out,
                 vidx, eidx, rows, xv_sh, semg, sems):
    c = lax.axis_index("c")
    s = lax.axis_index("s")
    pltpu.sync_copy(vv.at[s], vidx)
    pltpu.sync_copy(ee.at[s], eidx)
    r0 = s * VRT
    pltpu.sync_copy(zh, xv_sh.at[pl.ds(r0, VRT)])
    plsc.subcore_barrier()
    _pipelined_pass(xe2.at[c], eidx, vidx, rows, xv_sh, semg, sems)
    plsc.subcore_barrier()
    pltpu.sync_copy(xv_sh.at[pl.ds(r0, VRT)], xv_out.at[c].at[s])


def _sc_phase1(xsplit, v2d, e2d, zq, z16, ones16):
    return pl.kernel(
        _phase1_body,
        out_type=(jax.ShapeDtypeStruct((NC, NS, ERT, 2 * Q), jnp.float32),
                  jax.ShapeDtypeStruct((NC, NS, VRT, 16), jnp.float32)),
        mesh=plsc.VectorSubcoreMesh(**_MESH),
        compiler_params=_PARAMS,
        scratch_types=[
            pltpu.VMEM((RPT, CH), jnp.int32),
            pltpu.VMEM((RPT, CH), jnp.int32),
            pltpu.VMEM((2, CH, Q), jnp.float32),
            pltpu.VMEM((CH, 16), jnp.float32),
            pltpu.VMEM_SHARED((EN, Q), jnp.float32),
            pltpu.VMEM_SHARED((VN, 16), jnp.float32),
            pltpu.SemaphoreType.DMA((2,)),
            pltpu.SemaphoreType.DMA((2,)),
        ],
    )(xsplit, v2d, e2d, zq, z16, ones16)


def _sc_phase2(xe2, v2d, e2d, zh):
    return pl.kernel(
        _phase2_body,
        out_type=jax.ShapeDtypeStruct((NC, NS, VRT, 2 * Q), jnp.float32),
        mesh=plsc.VectorSubcoreMesh(**_MESH),
        compiler_params=_PARAMS,
        scratch_types=[
            pltpu.VMEM((RPT, CH), jnp.int32),
            pltpu.VMEM((RPT, CH), jnp.int32),
            pltpu.VMEM((2, CH, 2 * Q), jnp.float32),
            pltpu.VMEM_SHARED((VN, 2 * Q), jnp.float32),
            pltpu.SemaphoreType.DMA((2,)),
            pltpu.SemaphoreType.DMA((2,)),
        ],
    )(xe2, v2d, e2d, zh)


def _dense_body(xr, dr, v0r, v1r, wbr, war, wcr, bbr, bar, bcr,
                co, hlo, hro):
    deg = dr[0, :, 0:1] + dr[1, :, 0:1]
    a1 = xr[...] * deg
    a2 = jnp.concatenate([v0r[...], v1r[...]], axis=1)
    wb = wbr[...]
    wa = war[...]
    wc = wcr[...]
    f32 = jnp.float32
    c_ = (jnp.dot(a1, wb[:D], preferred_element_type=f32)
          + jnp.dot(a2, wb[D:], preferred_element_type=f32) + bbr[...])
    aa1 = jnp.abs(a1)
    aa2 = jnp.abs(a2)
    sl = (jnp.dot(aa1, wa[:D], preferred_element_type=f32)
          + jnp.dot(aa2, wa[D:], preferred_element_type=f32) + bar[...])
    sr = (jnp.dot(aa1, wc[:D], preferred_element_type=f32)
          + jnp.dot(aa2, wc[D:], preferred_element_type=f32) + bcr[...])
    co[...] = c_
    hlo[...] = c_ - sl
    hro[...] = c_ + sr


def _dense(X, dd, xv2, w_b, w_a, w_c, b_b, b_a, b_c):
    B = 1000
    grid = (N_NODES // B,)
    row_blk = pl.BlockSpec((B, D), lambda i: (i, 0))
    h_blk = pl.BlockSpec((B, 2 * Q), lambda i: (i, 0))
    w_blk = pl.BlockSpec((2 * D, D), lambda i: (0, 0))
    b_blk = pl.BlockSpec((1, D), lambda i: (0, 0))
    out_sd = jax.ShapeDtypeStruct((N_NODES, D), jnp.float32)
    return pl.pallas_call(
        _dense_body,
        grid=grid,
        in_specs=[
            row_blk,
            pl.BlockSpec((NC, B, 16), lambda i: (0, i, 0)),
            h_blk, h_blk,
            w_blk, w_blk, w_blk,
            b_blk, b_blk, b_blk,
        ],
        out_specs=(row_blk, row_blk, row_blk),
        out_shape=(out_sd, out_sd, out_sd),
    )(X, dd, xv2[0], xv2[1], w_b, w_a, w_c, b_b, b_a, b_c)


def kernel(X, vertex, edges, X0, w_b, w_a, w_c, b_b, b_a, b_c):
    del X0
    i32 = jnp.int32
    f32 = jnp.float32
    v = jnp.concatenate([vertex.astype(i32),
                         jnp.full((NNZP - NNZ,), VG, i32)])
    e = jnp.concatenate([edges.astype(i32),
                         jnp.full((NNZP - NNZ,), EG, i32)])
    Xp = jnp.concatenate([X, jnp.zeros((XN - N_NODES, D), f32)], axis=0)
    xsplit = jnp.stack([Xp[:, g * Q:(g + 1) * Q] for g in range(4)])
    v2d = v.reshape(NS, RPT, CH)
    e2d = e.reshape(NS, RPT, CH)
    zq = jnp.zeros((ERT // 2, Q), f32)
    z16 = jnp.zeros((VRT, 16), f32)
    zh = jnp.zeros((VRT, 2 * Q), f32)
    ones16 = jnp.ones((CH, 16), f32)
    xe, dd = _sc_phase1(xsplit, v2d, e2d, zq, z16, ones16)
    xe2 = xe.reshape(NC, EN, 2 * Q)
    xv = _sc_phase2(xe2, v2d, e2d, zh)
    xv2 = xv.reshape(NC, VN, 2 * Q)[:, :N_NODES]
    dd = dd.reshape(NC, VN, 16)[:, :N_NODES]
    return _dense(X, dd, xv2, w_b, w_a, w_c, b_b, b_a, b_c)


# speedup vs baseline: 1.4321x; 1.4321x over previous
import jax
import jax.numpy as jnp
from jax import lax
from jax.experimental import pallas as pl
from jax.experimental.pallas import tpu as pltpu
from jax.experimental.pallas import tpu_sc as plsc

N_NODES = 10000
N_HEDGES = 20000
NNZ = 320000
D = 128
Q = 32
NC = 2
NS = 16
CH = 80
RPT = NNZ // NS // CH
ERT = N_HEDGES // NS
VRT = N_NODES // NS

_MESH = dict(core_axis_name="c", subcore_axis_name="s", num_cores=NC,
             num_subcores=NS)
_PARAMS = pltpu.CompilerParams(use_tc_tiling_on_sc=False)


def _pipelined_pass(table, idx_g, idx_s, rows, acc, semg, sems, hook=None):
    pltpu.async_copy(table.at[idx_g.at[0]], rows.at[0], semg.at[0])

    def body(j, carry):
        nxt = j + 1

        @pl.when(nxt < RPT)
        def _():
            @pl.when(j >= 1)
            def _():
                pltpu.make_async_copy(
                    rows.at[nxt % 2], acc.at[idx_s.at[j - 1]],
                    sems.at[nxt % 2]).wait()

            pltpu.async_copy(table.at[idx_g.at[nxt]], rows.at[nxt % 2],
                             semg.at[nxt % 2])

        pltpu.make_async_copy(table.at[idx_g.at[j]], rows.at[j % 2],
                              semg.at[j % 2]).wait()
        pltpu.async_copy(rows.at[j % 2], acc.at[idx_s.at[j]],
                         sems.at[j % 2], add=True)
        if hook is not None:
            hook(j)
        return carry

    lax.fori_loop(0, RPT, body, 0)
    pltpu.make_async_copy(rows.at[0], acc.at[idx_s.at[RPT - 2]],
                          sems.at[0]).wait()
    pltpu.make_async_copy(rows.at[1], acc.at[idx_s.at[RPT - 1]],
                          sems.at[1]).wait()


def _phase1_body(xs, vv, ee, zq, z16, ones_h, xe_out, deg_out,
                 vidx, eidx, rows, ones_v, xe_sh, deg_sh, semg, sems, semd):
    c = lax.axis_index("c")
    s = lax.axis_index("s")
    pltpu.sync_copy(ones_h, ones_v)
    pltpu.sync_copy(vv.at[s], vidx)
    pltpu.sync_copy(ee.at[s], eidx)
    pltpu.sync_copy(z16, deg_sh.at[pl.ds(s * VRT, VRT)])
    r0 = s * ERT
    half = RPT // 2
    for k in range(2):
        g = 2 * c + k
        pltpu.sync_copy(zq, xe_sh.at[pl.ds(r0, VRT)])
        pltpu.sync_copy(zq, xe_sh.at[pl.ds(r0 + VRT, VRT)])
        plsc.subcore_barrier()

        def deg_hook(j):
            if k == 0:
                @pl.when(jnp.logical_and(j >= c * half, j < (c + 1) * half))
                def _():
                    pltpu.async_copy(ones_v, deg_sh.at[vidx.at[j]], semd,
                                     add=True)

        _pipelined_pass(xs.at[g], vidx, eidx, rows, xe_sh, semg, sems,
                        hook=deg_hook)
        if k == 0:
            def drain(j, carry):
                pltpu.make_async_copy(ones_v, deg_sh.at[vidx.at[0]],
                                      semd).wait()
                return carry

            lax.fori_loop(0, half, drain, 0)
        plsc.subcore_barrier()
        pltpu.sync_copy(xe_sh.at[pl.ds(r0, ERT)],
                        xe_out.at[c, s, :, pl.ds(k * Q, Q)])
    pltpu.sync_copy(deg_sh.at[pl.ds(s * VRT, VRT)], deg_out.at[c].at[s])


def _phase2_body(xe2, vv, ee, zh, xv_out,
                 vidx, eidx, rows, xv_sh, semg, sems):
    c = lax.axis_index("c")
    s = lax.axis_index("s")
    pltpu.sync_copy(vv.at[s], vidx)
    pltpu.sync_copy(ee.at[s], eidx)
    r0 = s * VRT
    pltpu.sync_copy(zh, xv_sh.at[pl.ds(r0, VRT)])
    plsc.subcore_barrier()
    _pipelined_pass(xe2.at[c], eidx, vidx, rows, xv_sh, semg, sems)
    plsc.subcore_barrier()
    pltpu.sync_copy(xv_sh.at[pl.ds(r0, VRT)], xv_out.at[c].at[s])


def _sc_phase1(xsplit, v2d, e2d, zq, z16, ones16):
    return pl.kernel(
        _phase1_body,
        out_type=(jax.ShapeDtypeStruct((NC, NS, ERT, 2 * Q), jnp.float32),
                  jax.ShapeDtypeStruct((NC, NS, VRT, 16), jnp.float32)),
        mesh=plsc.VectorSubcoreMesh(**_MESH),
        compiler_params=_PARAMS,
        scratch_types=[
            pltpu.VMEM((RPT, CH), jnp.int32),
            pltpu.VMEM((RPT, CH), jnp.int32),
            pltpu.VMEM((2, CH, Q), jnp.float32),
            pltpu.VMEM((CH, 16), jnp.float32),
            pltpu.VMEM_SHARED((N_HEDGES, Q), jnp.float32),
            pltpu.VMEM_SHARED((N_NODES, 16), jnp.float32),
            pltpu.SemaphoreType.DMA((2,)),
            pltpu.SemaphoreType.DMA((2,)),
            pltpu.SemaphoreType.DMA,
        ],
    )(xsplit, v2d, e2d, zq, z16, ones16)


def _sc_phase2(xe2, v2d, e2d, zh):
    return pl.kernel(
        _phase2_body,
        out_type=jax.ShapeDtypeStruct((NC, NS, VRT, 2 * Q), jnp.float32),
        mesh=plsc.VectorSubcoreMesh(**_MESH),
        compiler_params=_PARAMS,
        scratch_types=[
            pltpu.VMEM((RPT, CH), jnp.int32),
            pltpu.VMEM((RPT, CH), jnp.int32),
            pltpu.VMEM((2, CH, 2 * Q), jnp.float32),
            pltpu.VMEM_SHARED((N_NODES, 2 * Q), jnp.float32),
            pltpu.SemaphoreType.DMA((2,)),
            pltpu.SemaphoreType.DMA((2,)),
        ],
    )(xe2, v2d, e2d, zh)


def _dense_body(xr, dr, v0r, v1r, wbr, war, wcr, bbr, bar, bcr,
                co, hlo, hro):
    deg = dr[0, :, 0:1] + dr[1, :, 0:1]
    a1 = xr[...] * deg
    a2 = jnp.concatenate([v0r[...], v1r[...]], axis=1)
    wb = wbr[...]
    wa = war[...]
    wc = wcr[...]
    f32 = jnp.float32
    c_ = (jnp.dot(a1, wb[:D], preferred_element_type=f32)
          + jnp.dot(a2, wb[D:], preferred_element_type=f32) + bbr[...])
    aa1 = jnp.abs(a1)
    aa2 = jnp.abs(a2)
    sl = (jnp.dot(aa1, wa[:D], preferred_element_type=f32)
          + jnp.dot(aa2, wa[D:], preferred_element_type=f32) + bar[...])
    sr = (jnp.dot(aa1, wc[:D], preferred_element_type=f32)
          + jnp.dot(aa2, wc[D:], preferred_element_type=f32) + bcr[...])
    co[...] = c_
    hlo[...] = c_ - sl
    hro[...] = c_ + sr


def _dense(X, dd, xv2, w_b, w_a, w_c, b_b, b_a, b_c):
    B = 1000
    grid = (N_NODES // B,)
    row_blk = pl.BlockSpec((B, D), lambda i: (i, 0))
    h_blk = pl.BlockSpec((B, 2 * Q), lambda i: (i, 0))
    w_blk = pl.BlockSpec((2 * D, D), lambda i: (0, 0))
    b_blk = pl.BlockSpec((1, D), lambda i: (0, 0))
    out_sd = jax.ShapeDtypeStruct((N_NODES, D), jnp.float32)
    return pl.pallas_call(
        _dense_body,
        grid=grid,
        in_specs=[
            row_blk,
            pl.BlockSpec((NC, B, 16), lambda i: (0, i, 0)),
            h_blk, h_blk,
            w_blk, w_blk, w_blk,
            b_blk, b_blk, b_blk,
        ],
        out_specs=(row_blk, row_blk, row_blk),
        out_shape=(out_sd, out_sd, out_sd),
    )(X, dd, xv2[0], xv2[1], w_b, w_a, w_c, b_b, b_a, b_c)


def kernel(X, vertex, edges, X0, w_b, w_a, w_c, b_b, b_a, b_c):
    del X0
    v = vertex.astype(jnp.int32)
    e = edges.astype(jnp.int32)
    xsplit = jnp.stack([X[:, g * Q:(g + 1) * Q] for g in range(4)])
    v2d = v.reshape(NS, RPT, CH)
    e2d = e.reshape(NS, RPT, CH)
    zq = jnp.zeros((VRT, Q), jnp.float32)
    z16 = jnp.zeros((VRT, 16), jnp.float32)
    zh = jnp.zeros((VRT, 2 * Q), jnp.float32)
    ones16 = jnp.ones((CH, 16), jnp.float32)
    xe, dd = _sc_phase1(xsplit, v2d, e2d, zq, z16, ones16)
    xe2 = xe.reshape(NC, N_HEDGES, 2 * Q)
    xv = _sc_phase2(xe2, v2d, e2d, zh)
    xv2 = xv.reshape(NC, N_NODES, 2 * Q)
    dd = dd.reshape(NC, N_NODES, 16)
    return _dense(X, dd, xv2, w_b, w_a, w_c, b_b, b_a, b_c)


# speedup vs baseline: 2.0505x; 1.4319x over previous
import jax
import jax.numpy as jnp
from jax import lax
from jax.experimental import pallas as pl
from jax.experimental.pallas import tpu as pltpu
from jax.experimental.pallas import tpu_sc as plsc

N_NODES = 10000
N_HEDGES = 20000
NNZ = 320000
D = 128
Q = 32
NC = 2
NS = 16
CH = 80
RPT = NNZ // NS // CH
ERT = N_HEDGES // NS
VRT = N_NODES // NS

_MESH = dict(core_axis_name="c", subcore_axis_name="s", num_cores=NC,
             num_subcores=NS)
_PARAMS = pltpu.CompilerParams(use_tc_tiling_on_sc=False)


NBUF = 4


def _pipelined_pass(table, idx_g, idx_s, rows, acc, semg, sems, hook=None):
    for p in range(NBUF - 1):
        pltpu.async_copy(table.at[idx_g.at[p]], rows.at[p], semg.at[p])

    def body(j, carry):
        nxt = j + NBUF - 1

        @pl.when(nxt < RPT)
        def _():
            @pl.when(j >= 1)
            def _():
                pltpu.make_async_copy(
                    rows.at[nxt % NBUF], acc.at[idx_s.at[j - 1]],
                    sems.at[nxt % NBUF]).wait()

            pltpu.async_copy(table.at[idx_g.at[nxt]], rows.at[nxt % NBUF],
                             semg.at[nxt % NBUF])

        pltpu.make_async_copy(table.at[idx_g.at[j]], rows.at[j % NBUF],
                              semg.at[j % NBUF]).wait()
        pltpu.async_copy(rows.at[j % NBUF], acc.at[idx_s.at[j]],
                         sems.at[j % NBUF], add=True)
        if hook is not None:
            hook(j)
        return carry

    lax.fori_loop(0, RPT, body, 0)
    for p in range(NBUF):
        pltpu.make_async_copy(rows.at[p], acc.at[idx_s.at[RPT - 1]],
                              sems.at[p]).wait()


def _phase1_body(xs, vv, ee, zq, z16, ones_h, xe_out, deg_out,
                 vidx, eidx, rows, ones_v, xe_sh, deg_sh, semg, sems, semd):
    c = lax.axis_index("c")
    s = lax.axis_index("s")
    pltpu.sync_copy(ones_h, ones_v)
    pltpu.sync_copy(vv.at[s], vidx)
    pltpu.sync_copy(ee.at[s], eidx)
    pltpu.sync_copy(z16, deg_sh.at[pl.ds(s * VRT, VRT)])
    r0 = s * ERT
    half = RPT // 2
    for k in range(2):
        g = 2 * c + k
        pltpu.sync_copy(zq, xe_sh.at[pl.ds(r0, VRT)])
        pltpu.sync_copy(zq, xe_sh.at[pl.ds(r0 + VRT, VRT)])
        plsc.subcore_barrier()

        def deg_hook(j):
            if k == 0:
                @pl.when(jnp.logical_and(j >= c * half, j < (c + 1) * half))
                def _():
                    pltpu.async_copy(ones_v, deg_sh.at[vidx.at[j]], semd,
                                     add=True)

        _pipelined_pass(xs.at[g], vidx, eidx, rows, xe_sh, semg, sems,
                        hook=deg_hook)
        if k == 0:
            def drain(j, carry):
                pltpu.make_async_copy(ones_v, deg_sh.at[vidx.at[0]],
                                      semd).wait()
                return carry

            lax.fori_loop(0, half, drain, 0)
        plsc.subcore_barrier()
        pltpu.sync_copy(xe_sh.at[pl.ds(r0, ERT)],
                        xe_out.at[c, s, :, pl.ds(k * Q, Q)])
    pltpu.sync_copy(deg_sh.at[pl.ds(s * VRT, VRT)], deg_out.at[c].at[s])


def _phase2_body(xe2, vv, ee, zh, xv_out,
                 vidx, eidx, rows, xv_sh, semg, sems):
    c = lax.axis_index("c")
    s = lax.axis_index("s")
    pltpu.sync_copy(vv.at[s], vidx)
    pltpu.sync_copy(ee.at[s], eidx)
    r0 = s * VRT
    pltpu.sync_copy(zh, xv_sh.at[pl.ds(r0, VRT)])
    plsc.subcore_barrier()
    _pipelined_pass(xe2.at[c], eidx, vidx, rows, xv_sh, semg, sems)
    plsc.subcore_barrier()
    pltpu.sync_copy(xv_sh.at[pl.ds(r0, VRT)], xv_out.at[c].at[s])


def _sc_phase1(xsplit, v2d, e2d, zq, z16, ones16):
    return pl.kernel(
        _phase1_body,
        out_type=(jax.ShapeDtypeStruct((NC, NS, ERT, 2 * Q), jnp.float32),
                  jax.ShapeDtypeStruct((NC, NS, VRT, 16), jnp.float32)),
        mesh=plsc.VectorSubcoreMesh(**_MESH),
        compiler_params=_PARAMS,
        scratch_types=[
            pltpu.VMEM((RPT, CH), jnp.int32),
            pltpu.VMEM((RPT, CH), jnp.int32),
            pltpu.VMEM((NBUF, CH, Q), jnp.float32),
            pltpu.VMEM((CH, 16), jnp.float32),
            pltpu.VMEM_SHARED((N_HEDGES, Q), jnp.float32),
            pltpu.VMEM_SHARED((N_NODES, 16), jnp.float32),
            pltpu.SemaphoreType.DMA((NBUF,)),
            pltpu.SemaphoreType.DMA((NBUF,)),
            pltpu.SemaphoreType.DMA,
        ],
    )(xsplit, v2d, e2d, zq, z16, ones16)


def _sc_phase2(xe2, v2d, e2d, zh):
    return pl.kernel(
        _phase2_body,
        out_type=jax.ShapeDtypeStruct((NC, NS, VRT, 2 * Q), jnp.float32),
        mesh=plsc.VectorSubcoreMesh(**_MESH),
        compiler_params=_PARAMS,
        scratch_types=[
            pltpu.VMEM((RPT, CH), jnp.int32),
            pltpu.VMEM((RPT, CH), jnp.int32),
            pltpu.VMEM((NBUF, CH, 2 * Q), jnp.float32),
            pltpu.VMEM_SHARED((N_NODES, 2 * Q), jnp.float32),
            pltpu.SemaphoreType.DMA((NBUF,)),
            pltpu.SemaphoreType.DMA((NBUF,)),
        ],
    )(xe2, v2d, e2d, zh)


def _dense_body(xr, dr, v0r, v1r, wbr, war, wcr, bbr, bar, bcr,
                co, hlo, hro):
    deg = dr[0, :, 0:1] + dr[1, :, 0:1]
    a1 = xr[...] * deg
    a2 = jnp.concatenate([v0r[...], v1r[...]], axis=1)
    wb = wbr[...]
    wa = war[...]
    wc = wcr[...]
    f32 = jnp.float32
    c_ = (jnp.dot(a1, wb[:D], preferred_element_type=f32)
          + jnp.dot(a2, wb[D:], preferred_element_type=f32) + bbr[...])
    aa1 = jnp.abs(a1)
    aa2 = jnp.abs(a2)
    sl = (jnp.dot(aa1, wa[:D], preferred_element_type=f32)
          + jnp.dot(aa2, wa[D:], preferred_element_type=f32) + bar[...])
    sr = (jnp.dot(aa1, wc[:D], preferred_element_type=f32)
          + jnp.dot(aa2, wc[D:], preferred_element_type=f32) + bcr[...])
    co[...] = c_
    hlo[...] = c_ - sl
    hro[...] = c_ + sr


def _dense(X, dd, xv2, w_b, w_a, w_c, b_b, b_a, b_c):
    B = 1000
    grid = (N_NODES // B,)
    row_blk = pl.BlockSpec((B, D), lambda i: (i, 0))
    h_blk = pl.BlockSpec((B, 2 * Q), lambda i: (i, 0))
    w_blk = pl.BlockSpec((2 * D, D), lambda i: (0, 0))
    b_blk = pl.BlockSpec((1, D), lambda i: (0, 0))
    out_sd = jax.ShapeDtypeStruct((N_NODES, D), jnp.float32)
    return pl.pallas_call(
        _dense_body,
        grid=grid,
        in_specs=[
            row_blk,
            pl.BlockSpec((NC, B, 16), lambda i: (0, i, 0)),
            h_blk, h_blk,
            w_blk, w_blk, w_blk,
            b_blk, b_blk, b_blk,
        ],
        out_specs=(row_blk, row_blk, row_blk),
        out_shape=(out_sd, out_sd, out_sd),
    )(X, dd, xv2[0], xv2[1], w_b, w_a, w_c, b_b, b_a, b_c)


def kernel(X, vertex, edges, X0, w_b, w_a, w_c, b_b, b_a, b_c):
    del X0
    v = vertex.astype(jnp.int32)
    e = edges.astype(jnp.int32)
    xsplit = jnp.stack([X[:, g * Q:(g + 1) * Q] for g in range(4)])
    v2d = v.reshape(NS, RPT, CH)
    e2d = e.reshape(NS, RPT, CH)
    zq = jnp.zeros((VRT, Q), jnp.float32)
    z16 = jnp.zeros((VRT, 16), jnp.float32)
    zh = jnp.zeros((VRT, 2 * Q), jnp.float32)
    ones16 = jnp.ones((CH, 16), jnp.float32)
    xe, dd = _sc_phase1(xsplit, v2d, e2d, zq, z16, ones16)
    xe2 = xe.reshape(NC, N_HEDGES, 2 * Q)
    xv = _sc_phase2(xe2, v2d, e2d, zh)
    xv2 = xv.reshape(NC, N_NODES, 2 * Q)
    dd = dd.reshape(NC, N_NODES, 16)
    return _dense(X, dd, xv2, w_b, w_a, w_c, b_b, b_a, b_c)


# speedup vs baseline: 2.2422x; 1.0935x over previous
import jax
import jax.numpy as jnp
from jax import lax
from jax.experimental import pallas as pl
from jax.experimental.pallas import tpu as pltpu
from jax.experimental.pallas import tpu_sc as plsc

N_NODES = 10000
N_HEDGES = 20000
NNZ = 320000
D = 128
Q = 32
NC = 2
NS = 16
CH = 80
RPT = NNZ // NS // CH
ERT = N_HEDGES // NS
VRT = N_NODES // NS

_MESH = dict(core_axis_name="c", subcore_axis_name="s", num_cores=NC,
             num_subcores=NS)
_PARAMS = pltpu.CompilerParams(use_tc_tiling_on_sc=False)


NBUF = 8


def _pipelined_pass(table, idx_g, idx_s, rows, acc, semg, sems, hook=None):
    for p in range(NBUF - 1):
        pltpu.async_copy(table.at[idx_g.at[p]], rows.at[p], semg.at[p])

    def body(j, carry):
        nxt = j + NBUF - 1

        @pl.when(nxt < RPT)
        def _():
            @pl.when(j >= 1)
            def _():
                pltpu.make_async_copy(
                    rows.at[nxt % NBUF], acc.at[idx_s.at[j - 1]],
                    sems.at[nxt % NBUF]).wait()

            pltpu.async_copy(table.at[idx_g.at[nxt]], rows.at[nxt % NBUF],
                             semg.at[nxt % NBUF])

        pltpu.make_async_copy(table.at[idx_g.at[j]], rows.at[j % NBUF],
                              semg.at[j % NBUF]).wait()
        pltpu.async_copy(rows.at[j % NBUF], acc.at[idx_s.at[j]],
                         sems.at[j % NBUF], add=True)
        if hook is not None:
            hook(j)
        return carry

    lax.fori_loop(0, RPT, body, 0)
    for p in range(NBUF):
        pltpu.make_async_copy(rows.at[p], acc.at[idx_s.at[RPT - 1]],
                              sems.at[p]).wait()


def _phase1_body(xs, vv, ee, zq, z16, ones_h, xe_out, deg_out,
                 vidx, eidx, rows, ones_v, xe_sh, deg_sh, semg, sems, semd):
    c = lax.axis_index("c")
    s = lax.axis_index("s")
    pltpu.sync_copy(ones_h, ones_v)
    pltpu.sync_copy(vv.at[s], vidx)
    pltpu.sync_copy(ee.at[s], eidx)
    pltpu.sync_copy(z16, deg_sh.at[pl.ds(s * VRT, VRT)])
    r0 = s * ERT
    half = RPT // 2
    for k in range(2):
        g = 2 * c + k
        pltpu.sync_copy(zq, xe_sh.at[pl.ds(r0, VRT)])
        pltpu.sync_copy(zq, xe_sh.at[pl.ds(r0 + VRT, VRT)])
        plsc.subcore_barrier()

        def deg_hook(j):
            if k == 0:
                @pl.when(jnp.logical_and(j >= c * half, j < (c + 1) * half))
                def _():
                    pltpu.async_copy(ones_v, deg_sh.at[vidx.at[j]], semd,
                                     add=True)

        _pipelined_pass(xs.at[g], vidx, eidx, rows, xe_sh, semg, sems,
                        hook=deg_hook)
        if k == 0:
            def drain(j, carry):
                pltpu.make_async_copy(ones_v, deg_sh.at[vidx.at[0]],
                                      semd).wait()
                return carry

            lax.fori_loop(0, half, drain, 0)
        plsc.subcore_barrier()
        pltpu.sync_copy(xe_sh.at[pl.ds(r0, ERT)],
                        xe_out.at[c, s, :, pl.ds(k * Q, Q)])
    pltpu.sync_copy(deg_sh.at[pl.ds(s * VRT, VRT)], deg_out.at[c].at[s])


def _phase2_body(xe2, vv, ee, zh, xv_out,
                 vidx, eidx, rows, xv_sh, semg, sems):
    c = lax.axis_index("c")
    s = lax.axis_index("s")
    pltpu.sync_copy(vv.at[s], vidx)
    pltpu.sync_copy(ee.at[s], eidx)
    r0 = s * VRT
    pltpu.sync_copy(zh, xv_sh.at[pl.ds(r0, VRT)])
    plsc.subcore_barrier()
    _pipelined_pass(xe2.at[c], eidx, vidx, rows, xv_sh, semg, sems)
    plsc.subcore_barrier()
    pltpu.sync_copy(xv_sh.at[pl.ds(r0, VRT)], xv_out.at[c].at[s])


def _sc_phase1(xsplit, v2d, e2d, zq, z16, ones16):
    return pl.kernel(
        _phase1_body,
        out_type=(jax.ShapeDtypeStruct((NC, NS, ERT, 2 * Q), jnp.float32),
                  jax.ShapeDtypeStruct((NC, NS, VRT, 16), jnp.float32)),
        mesh=plsc.VectorSubcoreMesh(**_MESH),
        compiler_params=_PARAMS,
        scratch_types=[
            pltpu.VMEM((RPT, CH), jnp.int32),
            pltpu.VMEM((RPT, CH), jnp.int32),
            pltpu.VMEM((NBUF, CH, Q), jnp.float32),
            pltpu.VMEM((CH, 16), jnp.float32),
            pltpu.VMEM_SHARED((N_HEDGES, Q), jnp.float32),
            pltpu.VMEM_SHARED((N_NODES, 16), jnp.float32),
            pltpu.SemaphoreType.DMA((NBUF,)),
            pltpu.SemaphoreType.DMA((NBUF,)),
            pltpu.SemaphoreType.DMA,
        ],
    )(xsplit, v2d, e2d, zq, z16, ones16)


def _sc_phase2(xe2, v2d, e2d, zh):
    return pl.kernel(
        _phase2_body,
        out_type=jax.ShapeDtypeStruct((NC, NS, VRT, 2 * Q), jnp.float32),
        mesh=plsc.VectorSubcoreMesh(**_MESH),
        compiler_params=_PARAMS,
        scratch_types=[
            pltpu.VMEM((RPT, CH), jnp.int32),
            pltpu.VMEM((RPT, CH), jnp.int32),
            pltpu.VMEM((NBUF, CH, 2 * Q), jnp.float32),
            pltpu.VMEM_SHARED((N_NODES, 2 * Q), jnp.float32),
            pltpu.SemaphoreType.DMA((NBUF,)),
            pltpu.SemaphoreType.DMA((NBUF,)),
        ],
    )(xe2, v2d, e2d, zh)


def _dense_body(xr, dr, v0r, v1r, wbr, war, wcr, bbr, bar, bcr,
                co, hlo, hro):
    deg = dr[0, :, 0:1] + dr[1, :, 0:1]
    a1 = xr[...] * deg
    a2 = jnp.concatenate([v0r[...], v1r[...]], axis=1)
    wb = wbr[...]
    wa = war[...]
    wc = wcr[...]
    f32 = jnp.float32
    c_ = (jnp.dot(a1, wb[:D], preferred_element_type=f32)
          + jnp.dot(a2, wb[D:], preferred_element_type=f32) + bbr[...])
    aa1 = jnp.abs(a1)
    aa2 = jnp.abs(a2)
    sl = (jnp.dot(aa1, wa[:D], preferred_element_type=f32)
          + jnp.dot(aa2, wa[D:], preferred_element_type=f32) + bar[...])
    sr = (jnp.dot(aa1, wc[:D], preferred_element_type=f32)
          + jnp.dot(aa2, wc[D:], preferred_element_type=f32) + bcr[...])
    co[...] = c_
    hlo[...] = c_ - sl
    hro[...] = c_ + sr


def _dense(X, dd, xv2, w_b, w_a, w_c, b_b, b_a, b_c):
    B = 1000
    grid = (N_NODES // B,)
    row_blk = pl.BlockSpec((B, D), lambda i: (i, 0))
    h_blk = pl.BlockSpec((B, 2 * Q), lambda i: (i, 0))
    w_blk = pl.BlockSpec((2 * D, D), lambda i: (0, 0))
    b_blk = pl.BlockSpec((1, D), lambda i: (0, 0))
    out_sd = jax.ShapeDtypeStruct((N_NODES, D), jnp.float32)
    return pl.pallas_call(
        _dense_body,
        grid=grid,
        in_specs=[
            row_blk,
            pl.BlockSpec((NC, B, 16), lambda i: (0, i, 0)),
            h_blk, h_blk,
            w_blk, w_blk, w_blk,
            b_blk, b_blk, b_blk,
        ],
        out_specs=(row_blk, row_blk, row_blk),
        out_shape=(out_sd, out_sd, out_sd),
    )(X, dd, xv2[0], xv2[1], w_b, w_a, w_c, b_b, b_a, b_c)


def kernel(X, vertex, edges, X0, w_b, w_a, w_c, b_b, b_a, b_c):
    del X0
    v = vertex.astype(jnp.int32)
    e = edges.astype(jnp.int32)
    xsplit = jnp.stack([X[:, g * Q:(g + 1) * Q] for g in range(4)])
    v2d = v.reshape(NS, RPT, CH)
    e2d = e.reshape(NS, RPT, CH)
    zq = jnp.zeros((VRT, Q), jnp.float32)
    z16 = jnp.zeros((VRT, 16), jnp.float32)
    zh = jnp.zeros((VRT, 2 * Q), jnp.float32)
    ones16 = jnp.ones((CH, 16), jnp.float32)
    xe, dd = _sc_phase1(xsplit, v2d, e2d, zq, z16, ones16)
    xe2 = xe.reshape(NC, N_HEDGES, 2 * Q)
    xv = _sc_phase2(xe2, v2d, e2d, zh)
    xv2 = xv.reshape(NC, N_NODES, 2 * Q)
    dd = dd.reshape(NC, N_NODES, 16)
    return _dense(X, dd, xv2, w_b, w_a, w_c, b_b, b_a, b_c)


# speedup vs baseline: 2.2451x; 1.0013x over previous
import jax
import jax.numpy as jnp
from jax import lax
from jax.experimental import pallas as pl
from jax.experimental.pallas import tpu as pltpu
from jax.experimental.pallas import tpu_sc as plsc

N_NODES = 10000
N_HEDGES = 20000
NNZ = 320000
D = 128
Q = 32
NC = 2
NS = 16
CH = 80
RPT = NNZ // NS // CH
ERT = N_HEDGES // NS
VRT = N_NODES // NS

_MESH = dict(core_axis_name="c", subcore_axis_name="s", num_cores=NC,
             num_subcores=NS)
_PARAMS = pltpu.CompilerParams(use_tc_tiling_on_sc=False)


NB1 = 12
NB2 = 8


def _pipelined_pass(table, idx_g, idx_s, rows, acc, semg, sems, nbuf,
                    hook=None):
    for p in range(nbuf - 1):
        pltpu.async_copy(table.at[idx_g.at[p]], rows.at[p], semg.at[p])

    def body(j, carry):
        nxt = j + nbuf - 1

        @pl.when(nxt < RPT)
        def _():
            @pl.when(j >= 1)
            def _():
                pltpu.make_async_copy(
                    rows.at[nxt % nbuf], acc.at[idx_s.at[j - 1]],
                    sems.at[nxt % nbuf]).wait()

            pltpu.async_copy(table.at[idx_g.at[nxt]], rows.at[nxt % nbuf],
                             semg.at[nxt % nbuf])

        pltpu.make_async_copy(table.at[idx_g.at[j]], rows.at[j % nbuf],
                              semg.at[j % nbuf]).wait()
        pltpu.async_copy(rows.at[j % nbuf], acc.at[idx_s.at[j]],
                         sems.at[j % nbuf], add=True)
        if hook is not None:
            hook(j)
        return carry

    lax.fori_loop(0, RPT, body, 0)
    for p in range(nbuf):
        pltpu.make_async_copy(rows.at[p], acc.at[idx_s.at[RPT - 1]],
                              sems.at[p]).wait()


def _phase1_body(xs, vv, ee, zq, z16, ones_h, xe_out, deg_out,
                 vidx, eidx, rows, ones_v, xe_sh, deg_sh, semg, sems, semd):
    c = lax.axis_index("c")
    s = lax.axis_index("s")
    pltpu.sync_copy(ones_h, ones_v)
    pltpu.sync_copy(vv.at[s], vidx)
    pltpu.sync_copy(ee.at[s], eidx)
    pltpu.sync_copy(z16, deg_sh.at[pl.ds(s * VRT, VRT)])
    r0 = s * ERT
    half = RPT // 2
    for k in range(2):
        g = 2 * c + k
        pltpu.sync_copy(zq, xe_sh.at[pl.ds(r0, VRT)])
        pltpu.sync_copy(zq, xe_sh.at[pl.ds(r0 + VRT, VRT)])
        plsc.subcore_barrier()

        def deg_hook(j):
            if k == 0:
                @pl.when(jnp.logical_and(j >= c * half, j < (c + 1) * half))
                def _():
                    pltpu.async_copy(ones_v, deg_sh.at[vidx.at[j]], semd,
                                     add=True)

        _pipelined_pass(xs.at[g], vidx, eidx, rows, xe_sh, semg, sems,
                        NB1, hook=deg_hook)
        if k == 0:
            def drain(j, carry):
                pltpu.make_async_copy(ones_v, deg_sh.at[vidx.at[0]],
                                      semd).wait()
                return carry

            lax.fori_loop(0, half, drain, 0)
        plsc.subcore_barrier()
        pltpu.sync_copy(xe_sh.at[pl.ds(r0, ERT)],
                        xe_out.at[c, s, :, pl.ds(k * Q, Q)])
    pltpu.sync_copy(deg_sh.at[pl.ds(s * VRT, VRT)], deg_out.at[c].at[s])


def _phase2_body(xe2, vv, ee, zh, xv_out,
                 vidx, eidx, rows, xv_sh, semg, sems):
    c = lax.axis_index("c")
    s = lax.axis_index("s")
    pltpu.sync_copy(vv.at[s], vidx)
    pltpu.sync_copy(ee.at[s], eidx)
    r0 = s * VRT
    pltpu.sync_copy(zh, xv_sh.at[pl.ds(r0, VRT)])
    plsc.subcore_barrier()
    _pipelined_pass(xe2.at[c], eidx, vidx, rows, xv_sh, semg, sems, NB2)
    plsc.subcore_barrier()
    pltpu.sync_copy(xv_sh.at[pl.ds(r0, VRT)], xv_out.at[c].at[s])


def _sc_phase1(xsplit, v2d, e2d, zq, z16, ones16):
    return pl.kernel(
        _phase1_body,
        out_type=(jax.ShapeDtypeStruct((NC, NS, ERT, 2 * Q), jnp.float32),
                  jax.ShapeDtypeStruct((NC, NS, VRT, 16), jnp.float32)),
        mesh=plsc.VectorSubcoreMesh(**_MESH),
        compiler_params=_PARAMS,
        scratch_types=[
            pltpu.VMEM((RPT, CH), jnp.int32),
            pltpu.VMEM((RPT, CH), jnp.int32),
            pltpu.VMEM((NB1, CH, Q), jnp.float32),
            pltpu.VMEM((CH, 16), jnp.float32),
            pltpu.VMEM_SHARED((N_HEDGES, Q), jnp.float32),
            pltpu.VMEM_SHARED((N_NODES, 16), jnp.float32),
            pltpu.SemaphoreType.DMA((NB1,)),
            pltpu.SemaphoreType.DMA((NB1,)),
            pltpu.SemaphoreType.DMA,
        ],
    )(xsplit, v2d, e2d, zq, z16, ones16)


def _sc_phase2(xe2, v2d, e2d, zh):
    return pl.kernel(
        _phase2_body,
        out_type=jax.ShapeDtypeStruct((NC, NS, VRT, 2 * Q), jnp.float32),
        mesh=plsc.VectorSubcoreMesh(**_MESH),
        compiler_params=_PARAMS,
        scratch_types=[
            pltpu.VMEM((RPT, CH), jnp.int32),
            pltpu.VMEM((RPT, CH), jnp.int32),
            pltpu.VMEM((NB2, CH, 2 * Q), jnp.float32),
            pltpu.VMEM_SHARED((N_NODES, 2 * Q), jnp.float32),
            pltpu.SemaphoreType.DMA((NB2,)),
            pltpu.SemaphoreType.DMA((NB2,)),
        ],
    )(xe2, v2d, e2d, zh)


def _dense_body(xr, dr, v0r, v1r, wbr, war, wcr, bbr, bar, bcr,
                co, hlo, hro):
    deg = dr[0, :, 0:1] + dr[1, :, 0:1]
    a1 = xr[...] * deg
    a2 = jnp.concatenate([v0r[...], v1r[...]], axis=1)
    wb = wbr[...]
    wa = war[...]
    wc = wcr[...]
    f32 = jnp.float32
    c_ = (jnp.dot(a1, wb[:D], preferred_element_type=f32)
          + jnp.dot(a2, wb[D:], preferred_element_type=f32) + bbr[...])
    aa1 = jnp.abs(a1)
    aa2 = jnp.abs(a2)
    sl = (jnp.dot(aa1, wa[:D], preferred_element_type=f32)
          + jnp.dot(aa2, wa[D:], preferred_element_type=f32) + bar[...])
    sr = (jnp.dot(aa1, wc[:D], preferred_element_type=f32)
          + jnp.dot(aa2, wc[D:], preferred_element_type=f32) + bcr[...])
    co[...] = c_
    hlo[...] = c_ - sl
    hro[...] = c_ + sr


def _dense(X, dd, xv2, w_b, w_a, w_c, b_b, b_a, b_c):
    B = 1000
    grid = (N_NODES // B,)
    row_blk = pl.BlockSpec((B, D), lambda i: (i, 0))
    h_blk = pl.BlockSpec((B, 2 * Q), lambda i: (i, 0))
    w_blk = pl.BlockSpec((2 * D, D), lambda i: (0, 0))
    b_blk = pl.BlockSpec((1, D), lambda i: (0, 0))
    out_sd = jax.ShapeDtypeStruct((N_NODES, D), jnp.float32)
    return pl.pallas_call(
        _dense_body,
        grid=grid,
        in_specs=[
            row_blk,
            pl.BlockSpec((NC, B, 16), lambda i: (0, i, 0)),
            h_blk, h_blk,
            w_blk, w_blk, w_blk,
            b_blk, b_blk, b_blk,
        ],
        out_specs=(row_blk, row_blk, row_blk),
        out_shape=(out_sd, out_sd, out_sd),
    )(X, dd, xv2[0], xv2[1], w_b, w_a, w_c, b_b, b_a, b_c)


def kernel(X, vertex, edges, X0, w_b, w_a, w_c, b_b, b_a, b_c):
    del X0
    v = vertex.astype(jnp.int32)
    e = edges.astype(jnp.int32)
    xsplit = jnp.stack([X[:, g * Q:(g + 1) * Q] for g in range(4)])
    v2d = v.reshape(NS, RPT, CH)
    e2d = e.reshape(NS, RPT, CH)
    zq = jnp.zeros((VRT, Q), jnp.float32)
    z16 = jnp.zeros((VRT, 16), jnp.float32)
    zh = jnp.zeros((VRT, 2 * Q), jnp.float32)
    ones16 = jnp.ones((CH, 16), jnp.float32)
    xe, dd = _sc_phase1(xsplit, v2d, e2d, zq, z16, ones16)
    xe2 = xe.reshape(NC, N_HEDGES, 2 * Q)
    xv = _sc_phase2(xe2, v2d, e2d, zh)
    xv2 = xv.reshape(NC, N_NODES, 2 * Q)
    dd = dd.reshape(NC, N_NODES, 16)
    return _dense(X, dd, xv2, w_b, w_a, w_c, b_b, b_a, b_c)
